# baseline (speedup 1.0000x reference)
import jax
import jax.numpy as jnp
from jax import lax
from jax.experimental import pallas as pl
from jax.experimental.pallas import tpu as pltpu
from jax.experimental.pallas import tpu_sc as plsc

_N = 10000
_E = 320000
_D = 128
_NC = 2
_NS = 16
_NT = _NC * _NS
_CH = 64
_NCH = 160
_EPT = _CH * _NCH
_EP = _EPT * _NT
_NP = 10240
_WIN = 16
_NW = _NCH // _WIN
_NBUF = 4
_BLK = 1000
_GRID = _N // _BLK


def _sc_deg_body(dst_hbm, deg_hbm, dst_v, zb, ob, deg_sp):
    c = lax.axis_index("c")
    s = lax.axis_index("s")
    wid = c * _NS + s
    zeros16 = jnp.zeros((16,), jnp.float32)
    ones16 = jnp.ones((16,), jnp.float32)
    for j in range(_CH // 16):
        zb[pl.ds(j * 16, 16)] = zeros16
        ob[pl.ds(j * 16, 16)] = ones16
    nz = 640 // _CH
    for j in range(nz):
        pltpu.sync_copy(zb, deg_sp.at[pl.ds((s * nz + j) * _CH, _CH)])
    pltpu.sync_copy(dst_hbm.at[wid], dst_v)
    plsc.subcore_barrier()

    def body(k, carry):
        pltpu.sync_copy(ob, deg_sp.at[dst_v.at[k]], add=True)
        return carry

    lax.fori_loop(0, _NCH, body, 0)
    plsc.subcore_barrier()
    pltpu.sync_copy(deg_sp.at[pl.ds(s * 640, 640)],
                    deg_hbm.at[c, pl.ds(s * 640, 640)])


def _sc_agg_body(h_hbm, src_hbm, dst_hbm, out_hbm,
                 src_v, dst_v, bufs, sems, agg_sp):
    c = lax.axis_index("c")
    s = lax.axis_index("s")
    wid = c * _NS + s
    zeros16 = jnp.zeros((16,), jnp.float32)

    def zbody(i, carry):
        for j in range(8):
            bufs[0][i, pl.ds(j * 16, 16)] = zeros16
        return carry

    lax.fori_loop(0, _CH, zbody, 0)
    nz = 640 // _CH
    for j in range(nz):
        pltpu.sync_copy(bufs[0], agg_sp.at[pl.ds((s * nz + j) * _CH, _CH)])
    plsc.subcore_barrier()

    def wbody(w, carry):
        pltpu.sync_copy(src_hbm.at[wid, pl.ds(w * _WIN, _WIN)], src_v)
        pltpu.sync_copy(dst_hbm.at[wid, pl.ds(w * _WIN, _WIN)], dst_v)
        for q in range(_NBUF - 1):
            pltpu.async_copy(h_hbm.at[src_v.at[q]], bufs[q], sems[q])

        def body(i, carry2):
            for q in range(_NBUF):
                k = _NBUF * i + q
                pltpu.make_async_copy(h_hbm.at[src_v.at[k]], bufs[q],
                                      sems[q]).wait()
                pltpu.sync_copy(bufs[q], agg_sp.at[dst_v.at[k]], add=True)

                @pl.when(k + _NBUF - 1 < _WIN)
                def _():
                    qn = (q + _NBUF - 1) % _NBUF
                    pltpu.async_copy(h_hbm.at[src_v.at[k + _NBUF - 1]],
                                     bufs[qn], sems[qn])
            return carry2

        lax.fori_loop(0, _WIN // _NBUF, body, 0)
        return carry

    lax.fori_loop(0, _NW, wbody, 0)
    plsc.subcore_barrier()
    pltpu.sync_copy(agg_sp.at[pl.ds(s * 640, 640)],
                    out_hbm.at[c, pl.ds(s * 640, 640)])


def _sc_mesh():
    return plsc.VectorSubcoreMesh(core_axis_name="c", subcore_axis_name="s",
                                  num_cores=_NC, num_subcores=_NS)


def _sc_deg(dst_r):
    return pl.kernel(
        _sc_deg_body,
        out_type=jax.ShapeDtypeStruct((_NC, _NP), jnp.float32),
        mesh=_sc_mesh(),
        scratch_types=[
            pltpu.VMEM((_NCH, _CH), jnp.int32),
            pltpu.VMEM((_CH,), jnp.float32),
            pltpu.VMEM((_CH,), jnp.float32),
            pltpu.VMEM_SHARED((_NP,), jnp.float32),
        ],
    )(dst_r)


def _sc_agg(h, src_r, dst_r):
    return pl.kernel(
        _sc_agg_body,
        out_type=jax.ShapeDtypeStruct((_NC, _NP, _D), jnp.float32),
        mesh=_sc_mesh(),
        scratch_types=[
            pltpu.VMEM((_WIN, _CH), jnp.int32),
            pltpu.VMEM((_WIN, _CH), jnp.int32),
            [pltpu.VMEM((_CH, _D), jnp.float32) for _ in range(_NBUF)],
            [pltpu.SemaphoreType.DMA for _ in range(_NBUF)],
            pltpu.VMEM_SHARED((_NP, _D), jnp.float32),
        ],
    )(h, src_r, dst_r)


def _mm_body(x_ref, w_ref, o_ref):
    o_ref[...] = jnp.dot(x_ref[...], w_ref[...],
                         preferred_element_type=jnp.float32)


def _tc_matmul(x, w):
    return pl.pallas_call(
        _mm_body,
        grid=(_GRID,),
        in_specs=[pl.BlockSpec((_BLK, _D), lambda i: (i, 0)),
                  pl.BlockSpec((_D, _D), lambda i: (0, 0))],
        out_specs=pl.BlockSpec((_BLK, _D), lambda i: (i, 0)),
        out_shape=jax.ShapeDtypeStruct((_N, _D), jnp.float32),
    )(x, w)


def _comb1_body(h_ref, d0_ref, d1_ref, h1p_ref, dinv_ref):
    dinv = lax.rsqrt(d0_ref[0] + d1_ref[0] + 1.0)
    dinv_ref[...] = dinv
    h1p_ref[...] = h_ref[...] * dinv


def _tc_combine1(h_raw, degp3):
    return pl.pallas_call(
        _comb1_body,
        grid=(_GRID,),
        in_specs=[pl.BlockSpec((_BLK, _D), lambda i: (i, 0)),
                  pl.BlockSpec((1, _BLK, 1), lambda i: (0, i, 0)),
                  pl.BlockSpec((1, _BLK, 1), lambda i: (1, i, 0))],
        out_specs=[pl.BlockSpec((_BLK, _D), lambda i: (i, 0)),
                   pl.BlockSpec((_BLK, 1), lambda i: (i, 0))],
        out_shape=[jax.ShapeDtypeStruct((_N, _D), jnp.float32),
                   jax.ShapeDtypeStruct((_N, 1), jnp.float32)],
    )(h_raw, degp3, degp3)


def _layer_body(a0_ref, a1_ref, h1p_ref, dinv_ref, b1_ref, g_ref, be_ref,
                mu_ref, var_ref, w2_ref, o_ref):
    z = (a0_ref[0] + a1_ref[0] + h1p_ref[...]) * dinv_ref[...] + b1_ref[...]
    z = (z - mu_ref[...]) * lax.rsqrt(var_ref[...] + 1e-5) * g_ref[...] \
        + be_ref[...]
    z = jnp.maximum(z, 0.0)
    o_ref[...] = jnp.dot(z, w2_ref[...],
                         preferred_element_type=jnp.float32) * dinv_ref[...]


def _tc_layer(agg, h1p, dinv, b1, gamma, beta, mu, var, w2):
    vec = pl.BlockSpec((1, _D), lambda i: (0, 0))
    return pl.pallas_call(
        _layer_body,
        grid=(_GRID,),
        in_specs=[pl.BlockSpec((1, _BLK, _D), lambda i: (0, i, 0)),
                  pl.BlockSpec((1, _BLK, _D), lambda i: (1, i, 0)),
                  pl.BlockSpec((_BLK, _D), lambda i: (i, 0)),
                  pl.BlockSpec((_BLK, 1), lambda i: (i, 0)),
                  vec, vec, vec, vec, vec,
                  pl.BlockSpec((_D, _D), lambda i: (0, 0))],
        out_specs=pl.BlockSpec((_BLK, _D), lambda i: (i, 0)),
        out_shape=jax.ShapeDtypeStruct((_N, _D), jnp.float32),
    )(agg, agg, h1p, dinv, b1, gamma, beta, mu, var, w2)


def _final_body(a0_ref, a1_ref, h2p_ref, dinv_ref, b2_ref, o_ref):
    z = (a0_ref[0] + a1_ref[0] + h2p_ref[...]) * dinv_ref[...] + b2_ref[...]
    m = jnp.max(z, axis=1, keepdims=True)
    lse = jnp.log(jnp.sum(jnp.exp(z - m), axis=1, keepdims=True)) + m
    o_ref[...] = z - lse


def _tc_final(agg, h2p, dinv, b2):
    return pl.pallas_call(
        _final_body,
        grid=(_GRID,),
        in_specs=[pl.BlockSpec((1, _BLK, _D), lambda i: (0, i, 0)),
                  pl.BlockSpec((1, _BLK, _D), lambda i: (1, i, 0)),
                  pl.BlockSpec((_BLK, _D), lambda i: (i, 0)),
                  pl.BlockSpec((_BLK, 1), lambda i: (i, 0)),
                  pl.BlockSpec((1, _D), lambda i: (0, 0))],
        out_specs=pl.BlockSpec((_BLK, _D), lambda i: (i, 0)),
        out_shape=jax.ShapeDtypeStruct((_N, _D), jnp.float32),
    )(agg, agg, h2p, dinv, b2)


def kernel(x, edge_index, W1, b1, W2, b2, gamma, beta, bn_mean, bn_var):
    src = edge_index[0].astype(jnp.int32)
    dst = edge_index[1].astype(jnp.int32)
    pad = _EP - _E
    padr = jnp.arange(pad, dtype=jnp.int32)
    src_r = jnp.concatenate([src, padr % 256]) \
        .reshape(_NCH, _NT, _CH).transpose(1, 0, 2)
    dst_r = jnp.concatenate([dst, _N + padr % (_NP - _N)]) \
        .reshape(_NCH, _NT, _CH).transpose(1, 0, 2)

    degp = _sc_deg(dst_r)
    h_raw = _tc_matmul(x, W1)
    h1p, dinv = _tc_combine1(h_raw, degp.reshape(_NC, _NP, 1))

    agg1 = _sc_agg(h1p, src_r, dst_r)
    h2p = _tc_layer(agg1, h1p, dinv, b1.reshape(1, _D), gamma.reshape(1, _D),
                    beta.reshape(1, _D), bn_mean.reshape(1, _D),
                    bn_var.reshape(1, _D), W2)

    agg2 = _sc_agg(h2p, src_r, dst_r)
    return _tc_final(agg2, h2p, dinv, b2.reshape(1, _D))


# speedup vs baseline: 1.0878x; 1.0878x over previous
import jax
import jax.numpy as jnp
from jax import lax
from jax.experimental import pallas as pl
from jax.experimental.pallas import tpu as pltpu
from jax.experimental.pallas import tpu_sc as plsc

_N = 10000
_E = 320000
_D = 128
_NC = 2
_NS = 16
_NT = _NC * _NS
_CH = 128
_NCH = 80
_EPT = _CH * _NCH
_EP = _EPT * _NT
_NP = 10240
_WIN = 16
_NW = _NCH // _WIN
_BLK = 1000
_GRID = _N // _BLK


def _sc_deg_body(edges_hbm, deg_hbm, dst_v, zb, ob, deg_sp):
    c = lax.axis_index("c")
    s = lax.axis_index("s")
    wid = c * _NS + s
    zeros16 = jnp.zeros((16,), jnp.float32)
    ones16 = jnp.ones((16,), jnp.float32)
    for j in range(8):
        zb[pl.ds(j * 16, 16)] = zeros16
        ob[pl.ds(j * 16, 16)] = ones16
    for j in range(5):
        pltpu.sync_copy(zb, deg_sp.at[pl.ds((s * 5 + j) * _CH, _CH)])
    pltpu.sync_copy(edges_hbm.at[1, wid], dst_v)
    plsc.subcore_barrier()

    def body(k, carry):
        pltpu.sync_copy(ob, deg_sp.at[dst_v.at[k]], add=True)
        return carry

    lax.fori_loop(0, _NCH, body, 0)
    plsc.subcore_barrier()
    pltpu.sync_copy(deg_sp.at[pl.ds(s * 640, 640)],
                    deg_hbm.at[c, pl.ds(s * 640, 640)])


def _sc_agg_body(h_hbm, edges_hbm, out_hbm,
                 src_v, dst_v, buf_a, buf_b, agg_sp, sem_a, sem_b):
    c = lax.axis_index("c")
    s = lax.axis_index("s")
    wid = c * _NS + s
    zeros16 = jnp.zeros((16,), jnp.float32)

    def zbody(i, carry):
        for j in range(8):
            buf_a[i, pl.ds(j * 16, 16)] = zeros16
        return carry

    lax.fori_loop(0, _CH, zbody, 0)
    for j in range(5):
        pltpu.sync_copy(buf_a, agg_sp.at[pl.ds((s * 5 + j) * _CH, _CH)])
    plsc.subcore_barrier()

    def wbody(w, carry):
        pltpu.sync_copy(edges_hbm.at[0, wid, pl.ds(w * _WIN, _WIN)], src_v)
        pltpu.sync_copy(edges_hbm.at[1, wid, pl.ds(w * _WIN, _WIN)], dst_v)
        pltpu.async_copy(h_hbm.at[src_v.at[0]], buf_a, sem_a)

        def body(i, carry2):
            k0 = 2 * i
            k1 = k0 + 1
            pltpu.async_copy(h_hbm.at[src_v.at[k1]], buf_b, sem_b)
            pltpu.make_async_copy(h_hbm.at[src_v.at[k0]], buf_a, sem_a).wait()
            pltpu.sync_copy(buf_a, agg_sp.at[dst_v.at[k0]], add=True)

            @pl.when(i < _WIN // 2 - 1)
            def _():
                pltpu.async_copy(h_hbm.at[src_v.at[k0 + 2]], buf_a, sem_a)

            pltpu.make_async_copy(h_hbm.at[src_v.at[k1]], buf_b, sem_b).wait()
            pltpu.sync_copy(buf_b, agg_sp.at[dst_v.at[k1]], add=True)
            return carry2

        lax.fori_loop(0, _WIN // 2, body, 0)
        return carry

    lax.fori_loop(0, _NW, wbody, 0)
    plsc.subcore_barrier()
    pltpu.sync_copy(agg_sp.at[pl.ds(s * 640, 640)],
                    out_hbm.at[c, pl.ds(s * 640, 640)])


def _sc_mesh():
    return plsc.VectorSubcoreMesh(core_axis_name="c", subcore_axis_name="s",
                                  num_cores=_NC, num_subcores=_NS)


def _sc_deg(edges):
    return pl.kernel(
        _sc_deg_body,
        out_type=jax.ShapeDtypeStruct((_NC, _NP), jnp.float32),
        mesh=_sc_mesh(),
        scratch_types=[
            pltpu.VMEM((_NCH, _CH), jnp.int32),
            pltpu.VMEM((_CH,), jnp.float32),
            pltpu.VMEM((_CH,), jnp.float32),
            pltpu.VMEM_SHARED((_NP,), jnp.float32),
        ],
    )(edges)


def _sc_agg(h, edges):
    return pl.kernel(
        _sc_agg_body,
        out_type=jax.ShapeDtypeStruct((_NC, _NP, _D), jnp.float32),
        mesh=_sc_mesh(),
        scratch_types=[
            pltpu.VMEM((_WIN, _CH), jnp.int32),
            pltpu.VMEM((_WIN, _CH), jnp.int32),
            pltpu.VMEM((_CH, _D), jnp.float32),
            pltpu.VMEM((_CH, _D), jnp.float32),
            pltpu.VMEM_SHARED((_NP, _D), jnp.float32),
            pltpu.SemaphoreType.DMA,
            pltpu.SemaphoreType.DMA,
        ],
    )(h, edges)


def _mm_body(x_ref, w_ref, o_ref):
    o_ref[...] = jnp.dot(x_ref[...], w_ref[...],
                         preferred_element_type=jnp.float32)


def _tc_matmul(x, w):
    return pl.pallas_call(
        _mm_body,
        grid=(_GRID,),
        in_specs=[pl.BlockSpec((_BLK, _D), lambda i: (i, 0)),
                  pl.BlockSpec((_D, _D), lambda i: (0, 0))],
        out_specs=pl.BlockSpec((_BLK, _D), lambda i: (i, 0)),
        out_shape=jax.ShapeDtypeStruct((_N, _D), jnp.float32),
    )(x, w)


def _comb1_body(h_ref, d0_ref, d1_ref, h1p_ref, dinv_ref):
    dinv = lax.rsqrt(d0_ref[0] + d1_ref[0] + 1.0)
    dinv_ref[...] = dinv
    h1p_ref[...] = h_ref[...] * dinv


def _tc_combine1(h_raw, degp3):
    return pl.pallas_call(
        _comb1_body,
        grid=(_GRID,),
        in_specs=[pl.BlockSpec((_BLK, _D), lambda i: (i, 0)),
                  pl.BlockSpec((1, _BLK, 1), lambda i: (0, i, 0)),
                  pl.BlockSpec((1, _BLK, 1), lambda i: (1, i, 0))],
        out_specs=[pl.BlockSpec((_BLK, _D), lambda i: (i, 0)),
                   pl.BlockSpec((_BLK, 1), lambda i: (i, 0))],
        out_shape=[jax.ShapeDtypeStruct((_N, _D), jnp.float32),
                   jax.ShapeDtypeStruct((_N, 1), jnp.float32)],
    )(h_raw, degp3, degp3)


def _layer_body(a0_ref, a1_ref, h1p_ref, dinv_ref, b1_ref, g_ref, be_ref,
                mu_ref, var_ref, w2_ref, o_ref):
    z = (a0_ref[0] + a1_ref[0] + h1p_ref[...]) * dinv_ref[...] + b1_ref[...]
    z = (z - mu_ref[...]) * lax.rsqrt(var_ref[...] + 1e-5) * g_ref[...] \
        + be_ref[...]
    z = jnp.maximum(z, 0.0)
    o_ref[...] = jnp.dot(z, w2_ref[...],
                         preferred_element_type=jnp.float32) * dinv_ref[...]


def _tc_layer(agg, h1p, dinv, b1, gamma, beta, mu, var, w2):
    vec = pl.BlockSpec((1, _D), lambda i: (0, 0))
    return pl.pallas_call(
        _layer_body,
        grid=(_GRID,),
        in_specs=[pl.BlockSpec((1, _BLK, _D), lambda i: (0, i, 0)),
                  pl.BlockSpec((1, _BLK, _D), lambda i: (1, i, 0)),
                  pl.BlockSpec((_BLK, _D), lambda i: (i, 0)),
                  pl.BlockSpec((_BLK, 1), lambda i: (i, 0)),
                  vec, vec, vec, vec, vec,
                  pl.BlockSpec((_D, _D), lambda i: (0, 0))],
        out_specs=pl.BlockSpec((_BLK, _D), lambda i: (i, 0)),
        out_shape=jax.ShapeDtypeStruct((_N, _D), jnp.float32),
    )(agg, agg, h1p, dinv, b1, gamma, beta, mu, var, w2)


def _final_body(a0_ref, a1_ref, h2p_ref, dinv_ref, b2_ref, o_ref):
    z = (a0_ref[0] + a1_ref[0] + h2p_ref[...]) * dinv_ref[...] + b2_ref[...]
    m = jnp.max(z, axis=1, keepdims=True)
    lse = jnp.log(jnp.sum(jnp.exp(z - m), axis=1, keepdims=True)) + m
    o_ref[...] = z - lse


def _tc_final(agg, h2p, dinv, b2):
    return pl.pallas_call(
        _final_body,
        grid=(_GRID,),
        in_specs=[pl.BlockSpec((1, _BLK, _D), lambda i: (0, i, 0)),
                  pl.BlockSpec((1, _BLK, _D), lambda i: (1, i, 0)),
                  pl.BlockSpec((_BLK, _D), lambda i: (i, 0)),
                  pl.BlockSpec((_BLK, 1), lambda i: (i, 0)),
                  pl.BlockSpec((1, _D), lambda i: (0, 0))],
        out_specs=pl.BlockSpec((_BLK, _D), lambda i: (i, 0)),
        out_shape=jax.ShapeDtypeStruct((_N, _D), jnp.float32),
    )(agg, agg, h2p, dinv, b2)


def kernel(x, edge_index, W1, b1, W2, b2, gamma, beta, bn_mean, bn_var):
    pad = _EP - _E
    padr = jnp.arange(pad, dtype=jnp.int32)
    pads = jnp.stack([padr % 256, _N + padr % (_NP - _N)])
    edges = jnp.concatenate([edge_index.astype(jnp.int32), pads], axis=1) \
        .reshape(2, _NCH, _NT, _CH).transpose(0, 2, 1, 3)

    degp = _sc_deg(edges)
    h_raw = _tc_matmul(x, W1)
    h1p, dinv = _tc_combine1(h_raw, degp.reshape(_NC, _NP, 1))

    agg1 = _sc_agg(h1p, edges)
    h2p = _tc_layer(agg1, h1p, dinv, b1.reshape(1, _D), gamma.reshape(1, _D),
                    beta.reshape(1, _D), bn_mean.reshape(1, _D),
                    bn_var.reshape(1, _D), W2)

    agg2 = _sc_agg(h2p, edges)
    return _tc_final(agg2, h2p, dinv, b2.reshape(1, _D))


# speedup vs baseline: 1.1044x; 1.0153x over previous
import jax
import jax.numpy as jnp
from jax import lax
from jax.experimental import pallas as pl
from jax.experimental.pallas import tpu as pltpu
from jax.experimental.pallas import tpu_sc as plsc

_N = 10000
_E = 320000
_D = 128
_NC = 2
_NS = 16
_NT = _NC * _NS
_CH = 128
_NCH = 80
_EPT = _CH * _NCH
_EP = _EPT * _NT
_NP = 10240
_WIN = 16
_NW = _NCH // _WIN
_BLK = 2000
_GRID = _N // _BLK


def _sc_deg_body(edges_hbm, deg_hbm, dst_v, zb, ob, deg_sp):
    c = lax.axis_index("c")
    s = lax.axis_index("s")
    wid = c * _NS + s
    zeros16 = jnp.zeros((16,), jnp.float32)
    ones16 = jnp.ones((16,), jnp.float32)
    for j in range(8):
        zb[pl.ds(j * 16, 16)] = zeros16
        ob[pl.ds(j * 16, 16)] = ones16
    for j in range(5):
        pltpu.sync_copy(zb, deg_sp.at[pl.ds((s * 5 + j) * _CH, _CH)])
    pltpu.sync_copy(edges_hbm.at[1, wid], dst_v)
    plsc.subcore_barrier()

    def body(k, carry):
        pltpu.sync_copy(ob, deg_sp.at[dst_v.at[k]], add=True)
        return carry

    lax.fori_loop(0, _NCH, body, 0)
    plsc.subcore_barrier()
    pltpu.sync_copy(deg_sp.at[pl.ds(s * 640, 640)],
                    deg_hbm.at[c, pl.ds(s * 640, 640)])


def _sc_agg_body(h_hbm, edges_hbm, out_hbm,
                 src_v, dst_v, buf_a, buf_b, agg_sp, sem_a, sem_b):
    c = lax.axis_index("c")
    s = lax.axis_index("s")
    wid = c * _NS + s
    zeros16 = jnp.zeros((16,), jnp.float32)

    def zbody(i, carry):
        for j in range(8):
            buf_a[i, pl.ds(j * 16, 16)] = zeros16
        return carry

    lax.fori_loop(0, _CH, zbody, 0)
    for j in range(5):
        pltpu.sync_copy(buf_a, agg_sp.at[pl.ds((s * 5 + j) * _CH, _CH)])
    plsc.subcore_barrier()

    def wbody(w, carry):
        pltpu.sync_copy(edges_hbm.at[0, wid, pl.ds(w * _WIN, _WIN)], src_v)
        pltpu.sync_copy(edges_hbm.at[1, wid, pl.ds(w * _WIN, _WIN)], dst_v)
        pltpu.async_copy(h_hbm.at[src_v.at[0]], buf_a, sem_a)

        def body(i, carry2):
            k0 = 2 * i
            k1 = k0 + 1
            pltpu.async_copy(h_hbm.at[src_v.at[k1]], buf_b, sem_b)
            pltpu.make_async_copy(h_hbm.at[src_v.at[k0]], buf_a, sem_a).wait()
            pltpu.sync_copy(buf_a, agg_sp.at[dst_v.at[k0]], add=True)

            @pl.when(i < _WIN // 2 - 1)
            def _():
                pltpu.async_copy(h_hbm.at[src_v.at[k0 + 2]], buf_a, sem_a)

            pltpu.make_async_copy(h_hbm.at[src_v.at[k1]], buf_b, sem_b).wait()
            pltpu.sync_copy(buf_b, agg_sp.at[dst_v.at[k1]], add=True)
            return carry2

        lax.fori_loop(0, _WIN // 2, body, 0)
        return carry

    lax.fori_loop(0, _NW, wbody, 0)
    plsc.subcore_barrier()
    pltpu.sync_copy(agg_sp.at[pl.ds(s * 640, 640)],
                    out_hbm.at[c, pl.ds(s * 640, 640)])


def _sc_mesh():
    return plsc.VectorSubcoreMesh(core_axis_name="c", subcore_axis_name="s",
                                  num_cores=_NC, num_subcores=_NS)


def _sc_deg(edges):
    return pl.kernel(
        _sc_deg_body,
        out_type=jax.ShapeDtypeStruct((_NC, _NP), jnp.float32),
        mesh=_sc_mesh(),
        scratch_types=[
            pltpu.VMEM((_NCH, _CH), jnp.int32),
            pltpu.VMEM((_CH,), jnp.float32),
            pltpu.VMEM((_CH,), jnp.float32),
            pltpu.VMEM_SHARED((_NP,), jnp.float32),
        ],
    )(edges)


def _sc_agg(h, edges):
    return pl.kernel(
        _sc_agg_body,
        out_type=jax.ShapeDtypeStruct((_NC, _NP, _D), jnp.float32),
        mesh=_sc_mesh(),
        scratch_types=[
            pltpu.VMEM((_WIN, _CH), jnp.int32),
            pltpu.VMEM((_WIN, _CH), jnp.int32),
            pltpu.VMEM((_CH, _D), jnp.float32),
            pltpu.VMEM((_CH, _D), jnp.float32),
            pltpu.VMEM_SHARED((_NP, _D), jnp.float32),
            pltpu.SemaphoreType.DMA,
            pltpu.SemaphoreType.DMA,
        ],
    )(h, edges)


def _mm_body(x_ref, w_ref, o_ref):
    o_ref[...] = jnp.dot(x_ref[...], w_ref[...],
                         preferred_element_type=jnp.float32)


def _tc_matmul(x, w):
    return pl.pallas_call(
        _mm_body,
        grid=(_GRID,),
        in_specs=[pl.BlockSpec((_BLK, _D), lambda i: (i, 0)),
                  pl.BlockSpec((_D, _D), lambda i: (0, 0))],
        out_specs=pl.BlockSpec((_BLK, _D), lambda i: (i, 0)),
        out_shape=jax.ShapeDtypeStruct((_N, _D), jnp.float32),
    )(x, w)


def _comb1_body(h_ref, d0_ref, d1_ref, h1p_ref, dinv_ref):
    dinv = lax.rsqrt(d0_ref[0] + d1_ref[0] + 1.0)
    dinv_ref[...] = dinv
    h1p_ref[...] = h_ref[...] * dinv


def _tc_combine1(h_raw, degp3):
    return pl.pallas_call(
        _comb1_body,
        grid=(_GRID,),
        in_specs=[pl.BlockSpec((_BLK, _D), lambda i: (i, 0)),
                  pl.BlockSpec((1, _BLK, 1), lambda i: (0, i, 0)),
                  pl.BlockSpec((1, _BLK, 1), lambda i: (1, i, 0))],
        out_specs=[pl.BlockSpec((_BLK, _D), lambda i: (i, 0)),
                   pl.BlockSpec((_BLK, 1), lambda i: (i, 0))],
        out_shape=[jax.ShapeDtypeStruct((_N, _D), jnp.float32),
                   jax.ShapeDtypeStruct((_N, 1), jnp.float32)],
    )(h_raw, degp3, degp3)


def _layer_body(a0_ref, a1_ref, h1p_ref, dinv_ref, b1_ref, g_ref, be_ref,
                mu_ref, var_ref, w2_ref, o_ref):
    z = (a0_ref[0] + a1_ref[0] + h1p_ref[...]) * dinv_ref[...] + b1_ref[...]
    z = (z - mu_ref[...]) * lax.rsqrt(var_ref[...] + 1e-5) * g_ref[...] \
        + be_ref[...]
    z = jnp.maximum(z, 0.0)
    o_ref[...] = jnp.dot(z, w2_ref[...],
                         preferred_element_type=jnp.float32) * dinv_ref[...]


def _tc_layer(agg, h1p, dinv, b1, gamma, beta, mu, var, w2):
    vec = pl.BlockSpec((1, _D), lambda i: (0, 0))
    return pl.pallas_call(
        _layer_body,
        grid=(_GRID,),
        in_specs=[pl.BlockSpec((1, _BLK, _D), lambda i: (0, i, 0)),
                  pl.BlockSpec((1, _BLK, _D), lambda i: (1, i, 0)),
                  pl.BlockSpec((_BLK, _D), lambda i: (i, 0)),
                  pl.BlockSpec((_BLK, 1), lambda i: (i, 0)),
                  vec, vec, vec, vec, vec,
                  pl.BlockSpec((_D, _D), lambda i: (0, 0))],
        out_specs=pl.BlockSpec((_BLK, _D), lambda i: (i, 0)),
        out_shape=jax.ShapeDtypeStruct((_N, _D), jnp.float32),
    )(agg, agg, h1p, dinv, b1, gamma, beta, mu, var, w2)


def _final_body(a0_ref, a1_ref, h2p_ref, dinv_ref, b2_ref, o_ref):
    z = (a0_ref[0] + a1_ref[0] + h2p_ref[...]) * dinv_ref[...] + b2_ref[...]
    m = jnp.max(z, axis=1, keepdims=True)
    lse = jnp.log(jnp.sum(jnp.exp(z - m), axis=1, keepdims=True)) + m
    o_ref[...] = z - lse


def _tc_final(agg, h2p, dinv, b2):
    return pl.pallas_call(
        _final_body,
        grid=(_GRID,),
        in_specs=[pl.BlockSpec((1, _BLK, _D), lambda i: (0, i, 0)),
                  pl.BlockSpec((1, _BLK, _D), lambda i: (1, i, 0)),
                  pl.BlockSpec((_BLK, _D), lambda i: (i, 0)),
                  pl.BlockSpec((_BLK, 1), lambda i: (i, 0)),
                  pl.BlockSpec((1, _D), lambda i: (0, 0))],
        out_specs=pl.BlockSpec((_BLK, _D), lambda i: (i, 0)),
        out_shape=jax.ShapeDtypeStruct((_N, _D), jnp.float32),
    )(agg, agg, h2p, dinv, b2)


def kernel(x, edge_index, W1, b1, W2, b2, gamma, beta, bn_mean, bn_var):
    pad = _EP - _E
    padr = jnp.arange(pad, dtype=jnp.int32)
    pads = jnp.stack([padr % 256, _N + padr % (_NP - _N)])
    edges = jnp.concatenate([edge_index.astype(jnp.int32), pads], axis=1) \
        .reshape(2, _NCH, _NT, _CH).transpose(0, 2, 1, 3)

    degp = _sc_deg(edges)
    h_raw = _tc_matmul(x, W1)
    h1p, dinv = _tc_combine1(h_raw, degp.reshape(_NC, _NP, 1))

    agg1 = _sc_agg(h1p, edges)
    h2p = _tc_layer(agg1, h1p, dinv, b1.reshape(1, _D), gamma.reshape(1, _D),
                    beta.reshape(1, _D), bn_mean.reshape(1, _D),
                    bn_var.reshape(1, _D), W2)

    agg2 = _sc_agg(h2p, edges)
    return _tc_final(agg2, h2p, dinv, b2.reshape(1, _D))


# speedup vs baseline: 1.1448x; 1.0366x over previous
import jax
import jax.numpy as jnp
from jax import lax
from jax.experimental import pallas as pl
from jax.experimental.pallas import tpu as pltpu
from jax.experimental.pallas import tpu_sc as plsc

_N = 10000
_E = 320000
_D = 128
_NC = 2
_NS = 16
_NT = _NC * _NS
_CH = 128
_NCH = 80
_EPT = _CH * _NCH
_EP = _EPT * _NT
_NP = 10240
_WIN = 16
_NW = _NCH // _WIN
_BLK = 2000
_GRID = _N // _BLK


def _sc_deg_body(edges_hbm, deg_hbm, dst_v, zb, ob, deg_sp):
    c = lax.axis_index("c")
    s = lax.axis_index("s")
    wid = c * _NS + s
    zeros16 = jnp.zeros((16,), jnp.float32)
    ones16 = jnp.ones((16,), jnp.float32)
    for j in range(8):
        zb[pl.ds(j * 16, 16)] = zeros16
        ob[pl.ds(j * 16, 16)] = ones16
    for j in range(5):
        pltpu.sync_copy(zb, deg_sp.at[pl.ds((s * 5 + j) * _CH, _CH)])
    pltpu.sync_copy(edges_hbm.at[1, wid], dst_v)
    plsc.subcore_barrier()

    def body(k, carry):
        pltpu.sync_copy(ob, deg_sp.at[dst_v.at[k]], add=True)
        return carry

    lax.fori_loop(0, _NCH, body, 0)
    plsc.subcore_barrier()
    pltpu.sync_copy(deg_sp.at[pl.ds(s * 640, 640)],
                    deg_hbm.at[c, pl.ds(s * 640, 640)])


def _sc_agg_body(h_hbm, edges_hbm, out_hbm,
                 src_v, dst_v, buf_a, buf_b, agg_sp, sem_a, sem_b, sem_i):
    c = lax.axis_index("c")
    s = lax.axis_index("s")
    wid = c * _NS + s
    zeros16 = jnp.zeros((16,), jnp.float32)

    def zbody(i, carry):
        for j in range(8):
            buf_a[i, pl.ds(j * 16, 16)] = zeros16
        return carry

    lax.fori_loop(0, _CH, zbody, 0)
    for j in range(5):
        pltpu.sync_copy(buf_a, agg_sp.at[pl.ds((s * 5 + j) * _CH, _CH)])
    plsc.subcore_barrier()

    pltpu.async_copy(edges_hbm.at[0, wid, pl.ds(0, _WIN)], src_v.at[0], sem_i)
    pltpu.async_copy(edges_hbm.at[1, wid, pl.ds(0, _WIN)], dst_v.at[0], sem_i)

    def wbody(w, carry):
        p = lax.rem(w, 2)
        pltpu.make_async_copy(edges_hbm.at[0, wid, pl.ds(0, _WIN)],
                              src_v.at[p], sem_i).wait()
        pltpu.make_async_copy(edges_hbm.at[1, wid, pl.ds(0, _WIN)],
                              dst_v.at[p], sem_i).wait()

        @pl.when(w + 1 < _NW)
        def _():
            pn = lax.rem(w + 1, 2)
            pltpu.async_copy(edges_hbm.at[0, wid, pl.ds((w + 1) * _WIN, _WIN)],
                             src_v.at[pn], sem_i)
            pltpu.async_copy(edges_hbm.at[1, wid, pl.ds((w + 1) * _WIN, _WIN)],
                             dst_v.at[pn], sem_i)

        pltpu.async_copy(h_hbm.at[src_v.at[p, 0]], buf_a, sem_a)

        def body(i, carry2):
            k0 = 2 * i
            k1 = k0 + 1
            pltpu.async_copy(h_hbm.at[src_v.at[p, k1]], buf_b, sem_b)
            pltpu.make_async_copy(h_hbm.at[src_v.at[p, k0]], buf_a,
                                  sem_a).wait()
            pltpu.sync_copy(buf_a, agg_sp.at[dst_v.at[p, k0]], add=True)

            @pl.when(i < _WIN // 2 - 1)
            def _():
                pltpu.async_copy(h_hbm.at[src_v.at[p, k0 + 2]], buf_a, sem_a)

            pltpu.make_async_copy(h_hbm.at[src_v.at[p, k1]], buf_b,
                                  sem_b).wait()
            pltpu.sync_copy(buf_b, agg_sp.at[dst_v.at[p, k1]], add=True)
            return carry2

        lax.fori_loop(0, _WIN // 2, body, 0)
        return carry

    lax.fori_loop(0, _NW, wbody, 0)
    plsc.subcore_barrier()
    pltpu.sync_copy(agg_sp.at[pl.ds(s * 640, 640)],
                    out_hbm.at[c, pl.ds(s * 640, 640)])


def _sc_mesh():
    return plsc.VectorSubcoreMesh(core_axis_name="c", subcore_axis_name="s",
                                  num_cores=_NC, num_subcores=_NS)


def _sc_deg(edges):
    return pl.kernel(
        _sc_deg_body,
        out_type=jax.ShapeDtypeStruct((_NC, _NP), jnp.float32),
        mesh=_sc_mesh(),
        scratch_types=[
            pltpu.VMEM((_NCH, _CH), jnp.int32),
            pltpu.VMEM((_CH,), jnp.float32),
            pltpu.VMEM((_CH,), jnp.float32),
            pltpu.VMEM_SHARED((_NP,), jnp.float32),
        ],
    )(edges)


def _sc_agg(h, edges):
    return pl.kernel(
        _sc_agg_body,
        out_type=jax.ShapeDtypeStruct((_NC, _NP, _D), jnp.float32),
        mesh=_sc_mesh(),
        scratch_types=[
            pltpu.VMEM((2, _WIN, _CH), jnp.int32),
            pltpu.VMEM((2, _WIN, _CH), jnp.int32),
            pltpu.VMEM((_CH, _D), jnp.float32),
            pltpu.VMEM((_CH, _D), jnp.float32),
            pltpu.VMEM_SHARED((_NP, _D), jnp.float32),
            pltpu.SemaphoreType.DMA,
            pltpu.SemaphoreType.DMA,
            pltpu.SemaphoreType.DMA,
        ],
    )(h, edges)


def _mm_body(x_ref, w_ref, o_ref):
    o_ref[...] = jnp.dot(x_ref[...], w_ref[...],
                         preferred_element_type=jnp.float32)


def _tc_matmul(x, w):
    return pl.pallas_call(
        _mm_body,
        grid=(_GRID,),
        in_specs=[pl.BlockSpec((_BLK, _D), lambda i: (i, 0)),
                  pl.BlockSpec((_D, _D), lambda i: (0, 0))],
        out_specs=pl.BlockSpec((_BLK, _D), lambda i: (i, 0)),
        out_shape=jax.ShapeDtypeStruct((_N, _D), jnp.float32),
    )(x, w)


def _comb1_body(h_ref, d0_ref, d1_ref, h1p_ref, dinv_ref):
    dinv = lax.rsqrt(d0_ref[0] + d1_ref[0] + 1.0)
    dinv_ref[...] = dinv
    h1p_ref[...] = h_ref[...] * dinv


def _tc_combine1(h_raw, degp3):
    return pl.pallas_call(
        _comb1_body,
        grid=(_GRID,),
        in_specs=[pl.BlockSpec((_BLK, _D), lambda i: (i, 0)),
                  pl.BlockSpec((1, _BLK, 1), lambda i: (0, i, 0)),
                  pl.BlockSpec((1, _BLK, 1), lambda i: (1, i, 0))],
        out_specs=[pl.BlockSpec((_BLK, _D), lambda i: (i, 0)),
                   pl.BlockSpec((_BLK, 1), lambda i: (i, 0))],
        out_shape=[jax.ShapeDtypeStruct((_N, _D), jnp.float32),
                   jax.ShapeDtypeStruct((_N, 1), jnp.float32)],
    )(h_raw, degp3, degp3)


def _layer_body(a0_ref, a1_ref, h1p_ref, dinv_ref, b1_ref, g_ref, be_ref,
                mu_ref, var_ref, w2_ref, o_ref):
    z = (a0_ref[0] + a1_ref[0] + h1p_ref[...]) * dinv_ref[...] + b1_ref[...]
    z = (z - mu_ref[...]) * lax.rsqrt(var_ref[...] + 1e-5) * g_ref[...] \
        + be_ref[...]
    z = jnp.maximum(z, 0.0)
    o_ref[...] = jnp.dot(z, w2_ref[...],
                         preferred_element_type=jnp.float32) * dinv_ref[...]


def _tc_layer(agg, h1p, dinv, b1, gamma, beta, mu, var, w2):
    vec = pl.BlockSpec((1, _D), lambda i: (0, 0))
    return pl.pallas_call(
        _layer_body,
        grid=(_GRID,),
        in_specs=[pl.BlockSpec((1, _BLK, _D), lambda i: (0, i, 0)),
                  pl.BlockSpec((1, _BLK, _D), lambda i: (1, i, 0)),
                  pl.BlockSpec((_BLK, _D), lambda i: (i, 0)),
                  pl.BlockSpec((_BLK, 1), lambda i: (i, 0)),
                  vec, vec, vec, vec, vec,
                  pl.BlockSpec((_D, _D), lambda i: (0, 0))],
        out_specs=pl.BlockSpec((_BLK, _D), lambda i: (i, 0)),
        out_shape=jax.ShapeDtypeStruct((_N, _D), jnp.float32),
    )(agg, agg, h1p, dinv, b1, gamma, beta, mu, var, w2)


def _final_body(a0_ref, a1_ref, h2p_ref, dinv_ref, b2_ref, o_ref):
    z = (a0_ref[0] + a1_ref[0] + h2p_ref[...]) * dinv_ref[...] + b2_ref[...]
    m = jnp.max(z, axis=1, keepdims=True)
    lse = jnp.log(jnp.sum(jnp.exp(z - m), axis=1, keepdims=True)) + m
    o_ref[...] = z - lse


def _tc_final(agg, h2p, dinv, b2):
    return pl.pallas_call(
        _final_body,
        grid=(_GRID,),
        in_specs=[pl.BlockSpec((1, _BLK, _D), lambda i: (0, i, 0)),
                  pl.BlockSpec((1, _BLK, _D), lambda i: (1, i, 0)),
                  pl.BlockSpec((_BLK, _D), lambda i: (i, 0)),
                  pl.BlockSpec((_BLK, 1), lambda i: (i, 0)),
                  pl.BlockSpec((1, _D), lambda i: (0, 0))],
        out_specs=pl.BlockSpec((_BLK, _D), lambda i: (i, 0)),
        out_shape=jax.ShapeDtypeStruct((_N, _D), jnp.float32),
    )(agg, agg, h2p, dinv, b2)


def kernel(x, edge_index, W1, b1, W2, b2, gamma, beta, bn_mean, bn_var):
    pad = _EP - _E
    padr = jnp.arange(pad, dtype=jnp.int32)
    pads = jnp.stack([padr % 256, _N + padr % (_NP - _N)])
    edges = jnp.concatenate([edge_index.astype(jnp.int32), pads], axis=1) \
        .reshape(2, _NCH, _NT, _CH).transpose(0, 2, 1, 3)

    degp = _sc_deg(edges)
    h_raw = _tc_matmul(x, W1)
    h1p, dinv = _tc_combine1(h_raw, degp.reshape(_NC, _NP, 1))

    agg1 = _sc_agg(h1p, edges)
    h2p = _tc_layer(agg1, h1p, dinv, b1.reshape(1, _D), gamma.reshape(1, _D),
                    beta.reshape(1, _D), bn_mean.reshape(1, _D),
                    bn_var.reshape(1, _D), W2)

    agg2 = _sc_agg(h2p, edges)
    return _tc_final(agg2, h2p, dinv, b2.reshape(1, _D))


# speedup vs baseline: 1.1782x; 1.0292x over previous
import jax
import jax.numpy as jnp
from jax import lax
from jax.experimental import pallas as pl
from jax.experimental.pallas import tpu as pltpu
from jax.experimental.pallas import tpu_sc as plsc

_N = 10000
_E = 320000
_D = 128
_NC = 2
_NS = 16
_NT = _NC * _NS
_CH = 128
_NCH = 80
_EPT = _CH * _NCH
_EP = _EPT * _NT
_NP = 10240
_WIN = 16
_NW = _NCH // _WIN
_BLK = 2000
_GRID = _N // _BLK


def _sc_deg_body(edges_hbm, deg_hbm, dst_v, zb, ob, deg_sp):
    c = lax.axis_index("c")
    s = lax.axis_index("s")
    wid = c * _NS + s
    zeros16 = jnp.zeros((16,), jnp.float32)
    ones16 = jnp.ones((16,), jnp.float32)
    for j in range(8):
        zb[pl.ds(j * 16, 16)] = zeros16
        ob[pl.ds(j * 16, 16)] = ones16
    for j in range(5):
        pltpu.sync_copy(zb, deg_sp.at[pl.ds((s * 5 + j) * _CH, _CH)])
    pltpu.sync_copy(edges_hbm.at[1, wid], dst_v)
    plsc.subcore_barrier()

    def body(k, carry):
        pltpu.sync_copy(ob, deg_sp.at[dst_v.at[k]], add=True)
        return carry

    lax.fori_loop(0, _NCH, body, 0)
    plsc.subcore_barrier()
    pltpu.sync_copy(deg_sp.at[pl.ds(s * 640, 640)],
                    deg_hbm.at[c, pl.ds(s * 640, 640)])


def _sc_agg_body(h_hbm, edges_hbm, out_hbm,
                 src_v, dst_v, buf_a, buf_b, agg_sp, sem_a, sem_b, sem_i):
    c = lax.axis_index("c")
    s = lax.axis_index("s")
    wid = c * _NS + s
    zeros16 = jnp.zeros((16,), jnp.float32)

    def zbody(i, carry):
        for j in range(8):
            buf_a[i, pl.ds(j * 16, 16)] = zeros16
        return carry

    lax.fori_loop(0, _CH, zbody, 0)
    for j in range(5):
        pltpu.sync_copy(buf_a, agg_sp.at[pl.ds((s * 5 + j) * _CH, _CH)])
    plsc.subcore_barrier()

    pltpu.async_copy(edges_hbm.at[0, wid, pl.ds(0, _WIN)], src_v.at[0], sem_i)
    pltpu.async_copy(edges_hbm.at[1, wid, pl.ds(0, _WIN)], dst_v.at[0], sem_i)

    def wbody(w, carry):
        p = lax.rem(w, 2)
        pltpu.make_async_copy(edges_hbm.at[0, wid, pl.ds(0, _WIN)],
                              src_v.at[p], sem_i).wait()
        pltpu.make_async_copy(edges_hbm.at[1, wid, pl.ds(0, _WIN)],
                              dst_v.at[p], sem_i).wait()

        @pl.when(w + 1 < _NW)
        def _():
            pn = lax.rem(w + 1, 2)
            pltpu.async_copy(edges_hbm.at[0, wid, pl.ds((w + 1) * _WIN, _WIN)],
                             src_v.at[pn], sem_i)
            pltpu.async_copy(edges_hbm.at[1, wid, pl.ds((w + 1) * _WIN, _WIN)],
                             dst_v.at[pn], sem_i)

        pltpu.async_copy(h_hbm.at[src_v.at[p, 0]], buf_a, sem_a)

        def body(i, carry2):
            k0 = 2 * i
            k1 = k0 + 1
            pltpu.async_copy(h_hbm.at[src_v.at[p, k1]], buf_b, sem_b)
            pltpu.make_async_copy(h_hbm.at[src_v.at[p, k0]], buf_a,
                                  sem_a).wait()
            pltpu.sync_copy(buf_a, agg_sp.at[dst_v.at[p, k0]], add=True)

            @pl.when(i < _WIN // 2 - 1)
            def _():
                pltpu.async_copy(h_hbm.at[src_v.at[p, k0 + 2]], buf_a, sem_a)

            pltpu.make_async_copy(h_hbm.at[src_v.at[p, k1]], buf_b,
                                  sem_b).wait()
            pltpu.sync_copy(buf_b, agg_sp.at[dst_v.at[p, k1]], add=True)
            return carry2

        lax.fori_loop(0, _WIN // 2, body, 0)
        return carry

    lax.fori_loop(0, _NW, wbody, 0)
    plsc.subcore_barrier()
    pltpu.sync_copy(agg_sp.at[pl.ds(s * 640, 640)],
                    out_hbm.at[c, pl.ds(s * 640, 640)])


def _sc_mesh():
    return plsc.VectorSubcoreMesh(core_axis_name="c", subcore_axis_name="s",
                                  num_cores=_NC, num_subcores=_NS)


def _sc_deg(edges):
    return pl.kernel(
        _sc_deg_body,
        out_type=jax.ShapeDtypeStruct((_NC, _NP), jnp.float32),
        mesh=_sc_mesh(),
        scratch_types=[
            pltpu.VMEM((_NCH, _CH), jnp.int32),
            pltpu.VMEM((_CH,), jnp.float32),
            pltpu.VMEM((_CH,), jnp.float32),
            pltpu.VMEM_SHARED((_NP,), jnp.float32),
        ],
    )(edges)


def _sc_agg(h, edges):
    return pl.kernel(
        _sc_agg_body,
        out_type=jax.ShapeDtypeStruct((_NC, _NP, _D), jnp.float32),
        mesh=_sc_mesh(),
        scratch_types=[
            pltpu.VMEM((2, _WIN, _CH), jnp.int32),
            pltpu.VMEM((2, _WIN, _CH), jnp.int32),
            pltpu.VMEM((_CH, _D), jnp.float32),
            pltpu.VMEM((_CH, _D), jnp.float32),
            pltpu.VMEM_SHARED((_NP, _D), jnp.float32),
            pltpu.SemaphoreType.DMA,
            pltpu.SemaphoreType.DMA,
            pltpu.SemaphoreType.DMA,
        ],
    )(h, edges)


def _mm_body(x_ref, w_ref, o_ref):
    o_ref[...] = jnp.dot(x_ref[...], w_ref[...],
                         preferred_element_type=jnp.float32)


def _tc_matmul(x, w):
    return pl.pallas_call(
        _mm_body,
        grid=(_GRID,),
        in_specs=[pl.BlockSpec((_BLK, _D), lambda i: (i, 0)),
                  pl.BlockSpec((_D, _D), lambda i: (0, 0))],
        out_specs=pl.BlockSpec((_BLK, _D), lambda i: (i, 0)),
        out_shape=jax.ShapeDtypeStruct((_N, _D), jnp.float32),
    )(x, w)


_BLKC = 2560


def _comb1_body(h_ref, d_ref, h1p_ref, dinv_ref):
    dsum = d_ref[0:1, :] + d_ref[1:2, :]
    dinv = lax.rsqrt(jnp.swapaxes(dsum, 0, 1) + 1.0)
    dinv_ref[...] = dinv
    h1p_ref[...] = h_ref[...] * dinv


def _tc_combine1(h_raw, degp):
    return pl.pallas_call(
        _comb1_body,
        grid=(_NP // _BLKC,),
        in_specs=[pl.BlockSpec((_BLKC, _D), lambda i: (i, 0)),
                  pl.BlockSpec((_NC, _BLKC), lambda i: (0, i))],
        out_specs=[pl.BlockSpec((_BLKC, _D), lambda i: (i, 0)),
                   pl.BlockSpec((_BLKC, 1), lambda i: (i, 0))],
        out_shape=[jax.ShapeDtypeStruct((_N, _D), jnp.float32),
                   jax.ShapeDtypeStruct((_N, 1), jnp.float32)],
    )(h_raw, degp)


def _layer_body(a0_ref, a1_ref, h1p_ref, dinv_ref, b1_ref, g_ref, be_ref,
                mu_ref, var_ref, w2_ref, o_ref):
    z = (a0_ref[0] + a1_ref[0] + h1p_ref[...]) * dinv_ref[...] + b1_ref[...]
    z = (z - mu_ref[...]) * lax.rsqrt(var_ref[...] + 1e-5) * g_ref[...] \
        + be_ref[...]
    z = jnp.maximum(z, 0.0)
    o_ref[...] = jnp.dot(z, w2_ref[...],
                         preferred_element_type=jnp.float32) * dinv_ref[...]


def _tc_layer(agg, h1p, dinv, b1, gamma, beta, mu, var, w2):
    vec = pl.BlockSpec((1, _D), lambda i: (0, 0))
    return pl.pallas_call(
        _layer_body,
        grid=(_GRID,),
        in_specs=[pl.BlockSpec((1, _BLK, _D), lambda i: (0, i, 0)),
                  pl.BlockSpec((1, _BLK, _D), lambda i: (1, i, 0)),
                  pl.BlockSpec((_BLK, _D), lambda i: (i, 0)),
                  pl.BlockSpec((_BLK, 1), lambda i: (i, 0)),
                  vec, vec, vec, vec, vec,
                  pl.BlockSpec((_D, _D), lambda i: (0, 0))],
        out_specs=pl.BlockSpec((_BLK, _D), lambda i: (i, 0)),
        out_shape=jax.ShapeDtypeStruct((_N, _D), jnp.float32),
    )(agg, agg, h1p, dinv, b1, gamma, beta, mu, var, w2)


def _final_body(a0_ref, a1_ref, h2p_ref, dinv_ref, b2_ref, o_ref):
    z = (a0_ref[0] + a1_ref[0] + h2p_ref[...]) * dinv_ref[...] + b2_ref[...]
    m = jnp.max(z, axis=1, keepdims=True)
    lse = jnp.log(jnp.sum(jnp.exp(z - m), axis=1, keepdims=True)) + m
    o_ref[...] = z - lse


def _tc_final(agg, h2p, dinv, b2):
    return pl.pallas_call(
        _final_body,
        grid=(_GRID,),
        in_specs=[pl.BlockSpec((1, _BLK, _D), lambda i: (0, i, 0)),
                  pl.BlockSpec((1, _BLK, _D), lambda i: (1, i, 0)),
                  pl.BlockSpec((_BLK, _D), lambda i: (i, 0)),
                  pl.BlockSpec((_BLK, 1), lambda i: (i, 0)),
                  pl.BlockSpec((1, _D), lambda i: (0, 0))],
        out_specs=pl.BlockSpec((_BLK, _D), lambda i: (i, 0)),
        out_shape=jax.ShapeDtypeStruct((_N, _D), jnp.float32),
    )(agg, agg, h2p, dinv, b2)


def kernel(x, edge_index, W1, b1, W2, b2, gamma, beta, bn_mean, bn_var):
    pad = _EP - _E
    padr = jnp.arange(pad, dtype=jnp.int32)
    pads = jnp.stack([padr % 256, _N + padr % (_NP - _N)])
    edges = jnp.concatenate([edge_index.astype(jnp.int32), pads], axis=1) \
        .reshape(2, _NCH, _NT, _CH).transpose(0, 2, 1, 3)

    degp = _sc_deg(edges)
    h_raw = _tc_matmul(x, W1)
    h1p, dinv = _tc_combine1(h_raw, degp)

    agg1 = _sc_agg(h1p, edges)
    h2p = _tc_layer(agg1, h1p, dinv, b1.reshape(1, _D), gamma.reshape(1, _D),
                    beta.reshape(1, _D), bn_mean.reshape(1, _D),
                    bn_var.reshape(1, _D), W2)

    agg2 = _sc_agg(h2p, edges)
    return _tc_final(agg2, h2p, dinv, b2.reshape(1, _D))


# speedup vs baseline: 1.2102x; 1.0271x over previous
import jax
import jax.numpy as jnp
from jax import lax
from jax.experimental import pallas as pl
from jax.experimental.pallas import tpu as pltpu
from jax.experimental.pallas import tpu_sc as plsc

_N = 10000
_E = 320000
_D = 128
_NC = 2
_NS = 16
_NT = _NC * _NS
_CH = 128
_NCH = 80
_EPT = _CH * _NCH
_EP = _EPT * _NT
_NP = 10240
_WIN = 16
_NW = _NCH // _WIN
_BLK = 2000
_GRID = _N // _BLK


def _sc_deg_body(edges_hbm, deg_hbm, dst_v, zb, ob, deg_sp, sem_d):
    c = lax.axis_index("c")
    s = lax.axis_index("s")
    wid = c * _NS + s
    zeros16 = jnp.zeros((16,), jnp.float32)
    ones16 = jnp.ones((16,), jnp.float32)
    for j in range(8):
        zb[pl.ds(j * 16, 16)] = zeros16
        ob[pl.ds(j * 16, 16)] = ones16
    for j in range(5):
        pltpu.sync_copy(zb, deg_sp.at[pl.ds((s * 5 + j) * _CH, _CH)])
    pltpu.sync_copy(edges_hbm.at[1, wid], dst_v)
    plsc.subcore_barrier()

    def body(k, carry):
        pltpu.async_copy(ob, deg_sp.at[dst_v.at[k]], sem_d, add=True)
        return carry

    lax.fori_loop(0, _NCH, body, 0)

    def drain(k, carry):
        pltpu.make_async_copy(ob, deg_sp.at[dst_v.at[k]], sem_d).wait()
        return carry

    lax.fori_loop(0, _NCH, drain, 0)
    plsc.subcore_barrier()
    pltpu.sync_copy(deg_sp.at[pl.ds(s * 640, 640)],
                    deg_hbm.at[c, pl.ds(s * 640, 640)])


def _sc_agg_body(h_hbm, edges_hbm, out_hbm,
                 src_v, dst_v, buf_a, buf_b, agg_sp, sem_a, sem_b, sem_i):
    c = lax.axis_index("c")
    s = lax.axis_index("s")
    wid = c * _NS + s
    zeros16 = jnp.zeros((16,), jnp.float32)

    def zbody(i, carry):
        for j in range(8):
            buf_a[i, pl.ds(j * 16, 16)] = zeros16
        return carry

    lax.fori_loop(0, _CH, zbody, 0)
    for j in range(5):
        pltpu.sync_copy(buf_a, agg_sp.at[pl.ds((s * 5 + j) * _CH, _CH)])
    plsc.subcore_barrier()

    pltpu.async_copy(edges_hbm.at[0, wid, pl.ds(0, _WIN)], src_v.at[0], sem_i)
    pltpu.async_copy(edges_hbm.at[1, wid, pl.ds(0, _WIN)], dst_v.at[0], sem_i)

    def wbody(w, carry):
        p = lax.rem(w, 2)
        pltpu.make_async_copy(edges_hbm.at[0, wid, pl.ds(0, _WIN)],
                              src_v.at[p], sem_i).wait()
        pltpu.make_async_copy(edges_hbm.at[1, wid, pl.ds(0, _WIN)],
                              dst_v.at[p], sem_i).wait()

        @pl.when(w + 1 < _NW)
        def _():
            pn = lax.rem(w + 1, 2)
            pltpu.async_copy(edges_hbm.at[0, wid, pl.ds((w + 1) * _WIN, _WIN)],
                             src_v.at[pn], sem_i)
            pltpu.async_copy(edges_hbm.at[1, wid, pl.ds((w + 1) * _WIN, _WIN)],
                             dst_v.at[pn], sem_i)

        pltpu.async_copy(h_hbm.at[src_v.at[p, 0]], buf_a, sem_a)

        def body(i, carry2):
            k0 = 2 * i
            k1 = k0 + 1
            pltpu.async_copy(h_hbm.at[src_v.at[p, k1]], buf_b, sem_b)
            pltpu.make_async_copy(h_hbm.at[src_v.at[p, k0]], buf_a,
                                  sem_a).wait()
            pltpu.sync_copy(buf_a, agg_sp.at[dst_v.at[p, k0]], add=True)

            @pl.when(i < _WIN // 2 - 1)
            def _():
                pltpu.async_copy(h_hbm.at[src_v.at[p, k0 + 2]], buf_a, sem_a)

            pltpu.make_async_copy(h_hbm.at[src_v.at[p, k1]], buf_b,
                                  sem_b).wait()
            pltpu.sync_copy(buf_b, agg_sp.at[dst_v.at[p, k1]], add=True)
            return carry2

        lax.fori_loop(0, _WIN // 2, body, 0)
        return carry

    lax.fori_loop(0, _NW, wbody, 0)
    plsc.subcore_barrier()
    pltpu.sync_copy(agg_sp.at[pl.ds(s * 640, 640)],
                    out_hbm.at[c, pl.ds(s * 640, 640)])


def _sc_mesh():
    return plsc.VectorSubcoreMesh(core_axis_name="c", subcore_axis_name="s",
                                  num_cores=_NC, num_subcores=_NS)


def _sc_deg(edges):
    return pl.kernel(
        _sc_deg_body,
        out_type=jax.ShapeDtypeStruct((_NC, _NP), jnp.float32),
        mesh=_sc_mesh(),
        scratch_types=[
            pltpu.VMEM((_NCH, _CH), jnp.int32),
            pltpu.VMEM((_CH,), jnp.float32),
            pltpu.VMEM((_CH,), jnp.float32),
            pltpu.VMEM_SHARED((_NP,), jnp.float32),
            pltpu.SemaphoreType.DMA,
        ],
    )(edges)


def _sc_agg(h, edges):
    return pl.kernel(
        _sc_agg_body,
        out_type=jax.ShapeDtypeStruct((_NC, _NP, _D), jnp.float32),
        mesh=_sc_mesh(),
        scratch_types=[
            pltpu.VMEM((2, _WIN, _CH), jnp.int32),
            pltpu.VMEM((2, _WIN, _CH), jnp.int32),
            pltpu.VMEM((_CH, _D), jnp.float32),
            pltpu.VMEM((_CH, _D), jnp.float32),
            pltpu.VMEM_SHARED((_NP, _D), jnp.float32),
            pltpu.SemaphoreType.DMA,
            pltpu.SemaphoreType.DMA,
            pltpu.SemaphoreType.DMA,
        ],
    )(h, edges)


def _mm_body(x_ref, w_ref, o_ref):
    o_ref[...] = jnp.dot(x_ref[...], w_ref[...],
                         preferred_element_type=jnp.float32)


def _tc_matmul(x, w):
    return pl.pallas_call(
        _mm_body,
        grid=(_GRID,),
        in_specs=[pl.BlockSpec((_BLK, _D), lambda i: (i, 0)),
                  pl.BlockSpec((_D, _D), lambda i: (0, 0))],
        out_specs=pl.BlockSpec((_BLK, _D), lambda i: (i, 0)),
        out_shape=jax.ShapeDtypeStruct((_N, _D), jnp.float32),
    )(x, w)


_BLKC = 2560


def _comb1_body(h_ref, d_ref, h1p_ref, dinv_ref):
    dsum = d_ref[0:1, :] + d_ref[1:2, :]
    dinv = lax.rsqrt(jnp.swapaxes(dsum, 0, 1) + 1.0)
    dinv_ref[...] = dinv
    h1p_ref[...] = h_ref[...] * dinv


def _tc_combine1(h_raw, degp):
    return pl.pallas_call(
        _comb1_body,
        grid=(_NP // _BLKC,),
        in_specs=[pl.BlockSpec((_BLKC, _D), lambda i: (i, 0)),
                  pl.BlockSpec((_NC, _BLKC), lambda i: (0, i))],
        out_specs=[pl.BlockSpec((_BLKC, _D), lambda i: (i, 0)),
                   pl.BlockSpec((_BLKC, 1), lambda i: (i, 0))],
        out_shape=[jax.ShapeDtypeStruct((_N, _D), jnp.float32),
                   jax.ShapeDtypeStruct((_N, 1), jnp.float32)],
    )(h_raw, degp)


def _layer_body(a0_ref, a1_ref, h1p_ref, dinv_ref, b1_ref, g_ref, be_ref,
                mu_ref, var_ref, w2_ref, o_ref):
    z = (a0_ref[0] + a1_ref[0] + h1p_ref[...]) * dinv_ref[...] + b1_ref[...]
    z = (z - mu_ref[...]) * lax.rsqrt(var_ref[...] + 1e-5) * g_ref[...] \
        + be_ref[...]
    z = jnp.maximum(z, 0.0)
    o_ref[...] = jnp.dot(z, w2_ref[...],
                         preferred_element_type=jnp.float32) * dinv_ref[...]


def _tc_layer(agg, h1p, dinv, b1, gamma, beta, mu, var, w2):
    vec = pl.BlockSpec((1, _D), lambda i: (0, 0))
    return pl.pallas_call(
        _layer_body,
        grid=(_GRID,),
        in_specs=[pl.BlockSpec((1, _BLK, _D), lambda i: (0, i, 0)),
                  pl.BlockSpec((1, _BLK, _D), lambda i: (1, i, 0)),
                  pl.BlockSpec((_BLK, _D), lambda i: (i, 0)),
                  pl.BlockSpec((_BLK, 1), lambda i: (i, 0)),
                  vec, vec, vec, vec, vec,
                  pl.BlockSpec((_D, _D), lambda i: (0, 0))],
        out_specs=pl.BlockSpec((_BLK, _D), lambda i: (i, 0)),
        out_shape=jax.ShapeDtypeStruct((_N, _D), jnp.float32),
    )(agg, agg, h1p, dinv, b1, gamma, beta, mu, var, w2)


def _final_body(a0_ref, a1_ref, h2p_ref, dinv_ref, b2_ref, o_ref):
    z = (a0_ref[0] + a1_ref[0] + h2p_ref[...]) * dinv_ref[...] + b2_ref[...]
    m = jnp.max(z, axis=1, keepdims=True)
    lse = jnp.log(jnp.sum(jnp.exp(z - m), axis=1, keepdims=True)) + m
    o_ref[...] = z - lse


def _tc_final(agg, h2p, dinv, b2):
    return pl.pallas_call(
        _final_body,
        grid=(_GRID,),
        in_specs=[pl.BlockSpec((1, _BLK, _D), lambda i: (0, i, 0)),
                  pl.BlockSpec((1, _BLK, _D), lambda i: (1, i, 0)),
                  pl.BlockSpec((_BLK, _D), lambda i: (i, 0)),
                  pl.BlockSpec((_BLK, 1), lambda i: (i, 0)),
                  pl.BlockSpec((1, _D), lambda i: (0, 0))],
        out_specs=pl.BlockSpec((_BLK, _D), lambda i: (i, 0)),
        out_shape=jax.ShapeDtypeStruct((_N, _D), jnp.float32),
    )(agg, agg, h2p, dinv, b2)


def kernel(x, edge_index, W1, b1, W2, b2, gamma, beta, bn_mean, bn_var):
    pad = _EP - _E
    ppt = pad // _NT
    padr = jnp.arange(ppt, dtype=jnp.int32)
    pads = jnp.broadcast_to(
        jnp.stack([padr % 256, _N + padr])[:, None, :], (2, _NT, ppt))
    edges = jnp.concatenate(
        [edge_index.astype(jnp.int32).reshape(2, _NT, _E // _NT), pads],
        axis=2).reshape(2, _NT, _NCH, _CH)

    degp = _sc_deg(edges)
    h_raw = _tc_matmul(x, W1)
    h1p, dinv = _tc_combine1(h_raw, degp)

    agg1 = _sc_agg(h1p, edges)
    h2p = _tc_layer(agg1, h1p, dinv, b1.reshape(1, _D), gamma.reshape(1, _D),
                    beta.reshape(1, _D), bn_mean.reshape(1, _D),
                    bn_var.reshape(1, _D), W2)

    agg2 = _sc_agg(h2p, edges)
    return _tc_final(agg2, h2p, dinv, b2.reshape(1, _D))
